# baseline (speedup 1.0000x reference)
import functools

import jax
import jax.numpy as jnp
from jax import lax
from jax.experimental import pallas as pl
from jax.experimental.pallas import tpu as pltpu
from jax.experimental.pallas import tpu_sc as plsc

N_NODES = 10000
N_EDGES = 320000
NUM_FEATURES = 128
NUM_CLASSES = 10
HID = 8
IN_HEAD = 8

NPAD = 10240
BN = 256
NBLK = NPAD // BN
DUMMY = 10016

E_TOT = N_EDGES + N_NODES
NW = 32
UNROLL = 4
RPT = NPAD // 16
KSUB1, SG1 = 112, 1
KSUB2, SG2 = 96, 2


def _chunks_per_worker(k):
    return UNROLL * (-(-E_TOT // (k * NW * UNROLL)))

W1CAT = 80
W2CAT = 16


def _tc_proj1_body(x_ref, w_ref, as_ref, ad_ref, o_ref, o2_ref):
    h = jnp.dot(x_ref[...], w_ref[...], preferred_element_type=jnp.float32)
    a_src = jnp.dot(h, as_ref[...], preferred_element_type=jnp.float32)
    a_dst = jnp.dot(h, ad_ref[...], preferred_element_type=jnp.float32)
    z8 = jnp.zeros((BN, 8), jnp.float32)
    o_ref[...] = jnp.concatenate([h, a_src, z8], axis=1)
    o2_ref[...] = jnp.concatenate([a_dst, z8], axis=1)


def _tc_mid_body(p0_ref, p1_ref, e8_ref, b1_ref, w2e_ref, o_ref):
    acc = p0_ref[0] + p1_ref[0]
    num = acc[:, :64]
    den = jnp.dot(acc[:, 64:72], e8_ref[...],
                  preferred_element_type=jnp.float32)
    h1 = num / (den + 1e-16) + b1_ref[...]
    h1 = jnp.where(h1 > 0, h1, jnp.exp(jnp.minimum(h1, 0.0)) - 1.0)
    o_ref[...] = jnp.dot(h1, w2e_ref[...], preferred_element_type=jnp.float32)


def _tc_out_body(p0_ref, p1_ref, b2_ref, o_ref):
    acc = p0_ref[0] + p1_ref[0]
    z = acc[:, :10] / (acc[:, 10:11] + 1e-16) + b2_ref[...]
    m = jnp.max(z, axis=1, keepdims=True)
    zs = z - m
    o_ref[...] = zs - jnp.log(jnp.sum(jnp.exp(zs), axis=1, keepdims=True))


def _sc_edge_pass(ec_hbm, srows_hbm, atab_hbm, zeros_hbm, out_hbm,
                  accum, ebuf, atab, rows, orows, isem, tsem, grsem, ssem,
                  *, width, n_feat, n_head, asrc_col, bf16_tab, ksub, sg,
                  ch_per_w):
    cid = lax.axis_index("c")
    sid = lax.axis_index("s")
    wid = cid * 16 + sid
    row0 = sid * RPT
    ch = n_feat // n_head
    k = ksub * sg

    tcopy = pltpu.async_copy(atab_hbm, atab, tsem)
    pltpu.sync_copy(zeros_hbm, accum.at[pl.ds(row0, RPT)])
    tcopy.wait()
    plsc.subcore_barrier()

    zv = jnp.zeros((16,), jnp.float32)
    for b in range(2):
        for g in range(k // 16):
            eis = lax.iota(jnp.int32, 16) + g * 16
            for col in range(n_feat + n_head, width):
                plsc.store_scatter(
                    orows.at[b], [eis, jnp.full((16,), col, jnp.int32)], zv)

    def fetch(j, q):
        pltpu.async_copy(ec_hbm.at[wid * ch_per_w + j], ebuf.at[q],
                         isem.at[q])

    def wait_fetch(q):
        pltpu.make_async_copy(ec_hbm.at[0], ebuf.at[q], isem.at[q]).wait()

    def gather(b, q):
        for s in range(sg):
            pltpu.async_copy(srows_hbm.at[ebuf.at[q, 0, s]],
                             rows.at[b, pl.ds(s * ksub, ksub)], grsem.at[b])

    def wait_gather(b):
        for s in range(sg):
            pltpu.make_async_copy(srows_hbm.at[ebuf.at[0, 0, s]],
                                  rows.at[b, pl.ds(s * ksub, ksub)],
                                  grsem.at[b]).wait()

    def scatter(b, q):
        for s in range(sg):
            pltpu.async_copy(orows.at[b, pl.ds(s * ksub, ksub)],
                             accum.at[ebuf.at[q, 1, s]], ssem.at[b],
                             add=True)

    def wait_scatter(b):
        for s in range(sg):
            pltpu.make_async_copy(orows.at[b, pl.ds(s * ksub, ksub)],
                                  accum.at[ebuf.at[0, 1, s]],
                                  ssem.at[b]).wait()

    def compute(b, q):
        rb = rows.at[b]
        ob = orows.at[b]
        for s_ in range(sg):

            def group_body(g, inner, s_=s_):
                ei = lax.iota(jnp.int32, 16) + (s_ * ksub + g * 16)
                dstv = ebuf[q, 1, s_, pl.ds(g * 16, 16)]
                if bf16_tab:
                    dvs = []
                    dstv4 = dstv * (n_head // 2)
                    for pc in range(n_head // 2):
                        pv = plsc.load_gather(atab, [dstv4 + pc])
                        d0, d1 = plsc.unpack(
                            plsc.bitcast(pv, jnp.bfloat16),
                            format=plsc.PackFormat.INTERLEAVED,
                            preferred_element_type=jnp.float32)
                        dvs += [d0, d1]
                else:
                    dvs = [plsc.load_gather(atab, [dstv])]
                ws = []
                for hd in range(n_head):
                    av = plsc.load_gather(
                        rb, [ei, jnp.full((16,), asrc_col + hd, jnp.int32)])
                    s = av + dvs[hd]
                    w = jnp.exp(jnp.maximum(s, s * 0.2))
                    plsc.store_scatter(
                        ob, [ei, jnp.full((16,), n_feat + hd, jnp.int32)], w)
                    ws.append(w)
                for col in range(n_feat):
                    hv = plsc.load_gather(
                        rb, [ei, jnp.full((16,), col, jnp.int32)])
                    plsc.store_scatter(
                        ob, [ei, jnp.full((16,), col, jnp.int32)],
                        hv * ws[col // ch])
                return inner

            lax.fori_loop(0, ksub // 16, group_body, 0)

    def step(j, jq, with_scatter_wait):
        b = jq % 2
        q = jq % 4
        wait_gather(b)
        if with_scatter_wait:
            wait_scatter(b)
        fetch(j + 2, (jq + 2) % 4)
        wait_fetch((jq + 1) % 4)
        gather((b + 1) % 2, (jq + 1) % 4)
        compute(b, q)
        scatter(b, q)

    fetch(0, 0)
    fetch(1, 1)
    wait_fetch(0)
    gather(0, 0)
    for j in range(2):
        step(j, j, with_scatter_wait=False)
    for j in range(2, UNROLL):
        step(j, j, with_scatter_wait=True)

    def loop_body(m, carry):
        j0 = m * UNROLL
        for r in range(UNROLL):
            step(j0 + r, r, with_scatter_wait=True)
        return carry

    lax.fori_loop(1, ch_per_w // UNROLL, loop_body, 0)

    wait_scatter(0)
    wait_scatter(1)
    wait_gather(ch_per_w % 2)
    wait_fetch((ch_per_w + 1) % 4)
    plsc.subcore_barrier()
    pltpu.sync_copy(accum.at[pl.ds(row0, RPT)],
                    out_hbm.at[cid, pl.ds(row0, RPT)])


@functools.cache
def _make_sc_pass(width, n_feat, n_head, asrc_col, bf16_tab, ksub, sg):
    mesh = plsc.VectorSubcoreMesh(core_axis_name="c", subcore_axis_name="s")
    ch_per_w = _chunks_per_worker(ksub * sg)
    body = functools.partial(_sc_edge_pass, width=width, n_feat=n_feat,
                             n_head=n_head, asrc_col=asrc_col,
                             bf16_tab=bf16_tab, ksub=ksub, sg=sg,
                             ch_per_w=ch_per_w)
    k = ksub * sg
    tab_t = (pltpu.VMEM((NPAD * n_head // 2,), jnp.int32) if bf16_tab
             else pltpu.VMEM((NPAD,), jnp.float32))
    return pl.kernel(
        body,
        out_type=jax.ShapeDtypeStruct((2, NPAD, width), jnp.float32),
        mesh=mesh,
        compiler_params=pltpu.CompilerParams(
            needs_layout_passes=False, use_tc_tiling_on_sc=False),
        scratch_types=[
            pltpu.VMEM_SHARED((NPAD, width), jnp.float32),
            pltpu.VMEM((4, 2, sg, ksub), jnp.int32),
            tab_t,
            pltpu.VMEM((2, k, width), jnp.float32),
            pltpu.VMEM((2, k, width), jnp.float32),
            pltpu.SemaphoreType.DMA((4,)),
            pltpu.SemaphoreType.DMA,
            pltpu.SemaphoreType.DMA((2,)),
            pltpu.SemaphoreType.DMA((2,)),
        ],
    )


def _tc_call(body, grid, in_specs, out_specs, out_shape):
    return pl.pallas_call(body, grid=grid, in_specs=in_specs,
                          out_specs=out_specs, out_shape=out_shape)


def kernel(x, edge_index, W1, att_src1, att_dst1, b1, W2, att_src2,
           att_dst2, b2):
    f32 = jnp.float32

    loop = jnp.arange(N_NODES, dtype=jnp.int32)

    def make_ec(ksub, sg):
        k = ksub * sg
        nch_alloc = _chunks_per_worker(k) * NW + 2
        padi = jnp.full((nch_alloc * k - E_TOT,), DUMMY, jnp.int32)
        s = jnp.concatenate([edge_index[0].astype(jnp.int32), loop, padi])
        d = jnp.concatenate([edge_index[1].astype(jnp.int32), loop, padi])
        return jnp.stack([s.reshape(nch_alloc, sg, ksub),
                          d.reshape(nch_alloc, sg, ksub)], axis=1)

    ec1 = make_ec(KSUB1, SG1)
    ec2 = make_ec(KSUB2, SG2)

    ar = jnp.arange(IN_HEAD * HID)
    As = jnp.zeros((IN_HEAD * HID, IN_HEAD), f32).at[
        ar, ar // HID].set(att_src1.reshape(-1))
    Ad = jnp.zeros((IN_HEAD * HID, IN_HEAD), f32).at[
        ar, ar // HID].set(att_dst1.reshape(-1))
    E8 = jnp.zeros((IN_HEAD, IN_HEAD * HID), f32).at[
        ar // HID, ar].set(1.0)
    W2e = jnp.concatenate([
        W2, W2 @ att_src2.reshape(NUM_CLASSES, 1),
        W2 @ att_dst2.reshape(NUM_CLASSES, 1),
        jnp.zeros((IN_HEAD * HID, 4), f32)], axis=1)

    xp = jnp.zeros((NPAD, NUM_FEATURES), f32).at[:N_NODES].set(x)

    full = lambda shp: pl.BlockSpec(shp, lambda i: (0, 0))
    blk = lambda w: pl.BlockSpec((BN, w), lambda i: (i, 0))
    srows1, drows1 = _tc_call(
        _tc_proj1_body, (NBLK,),
        [blk(NUM_FEATURES), full((NUM_FEATURES, 64)), full((64, 8)),
         full((64, 8))],
        [blk(W1CAT), blk(16)],
        [jax.ShapeDtypeStruct((NPAD, W1CAT), f32),
         jax.ShapeDtypeStruct((NPAD, 16), f32)])(xp, W1, As, Ad)

    adst_bf = drows1[:, :IN_HEAD].astype(jnp.bfloat16)
    adst_pairs = jax.lax.bitcast_convert_type(
        adst_bf.reshape(NPAD, IN_HEAD // 2, 2), jnp.int32).reshape(-1)
    zeros1 = jnp.zeros((RPT, W1CAT), f32)
    part1 = _make_sc_pass(W1CAT, 64, IN_HEAD, 64, True, KSUB1, SG1)(
        ec1, srows1, adst_pairs, zeros1)

    b1r = b1.reshape(1, 64)
    rows2 = _tc_call(
        _tc_mid_body, (NBLK,),
        [pl.BlockSpec((1, BN, W1CAT), lambda i: (0, i, 0)),
         pl.BlockSpec((1, BN, W1CAT), lambda i: (1, i, 0)),
         full((IN_HEAD, 64)), full((1, 64)), full((64, W2CAT))],
        blk(W2CAT),
        jax.ShapeDtypeStruct((NPAD, W2CAT), f32))(
            part1, part1, E8, b1r, W2e)

    adst2 = rows2[:, 11]
    zeros2 = jnp.zeros((RPT, W2CAT), f32)
    part2 = _make_sc_pass(W2CAT, 10, 1, 10, False, KSUB2, SG2)(
        ec2, rows2, adst2, zeros2)

    b2r = b2.reshape(1, NUM_CLASSES)
    out = _tc_call(
        _tc_out_body, (NBLK,),
        [pl.BlockSpec((1, BN, W2CAT), lambda i: (0, i, 0)),
         pl.BlockSpec((1, BN, W2CAT), lambda i: (1, i, 0)),
         full((1, NUM_CLASSES))],
        blk(NUM_CLASSES),
        jax.ShapeDtypeStruct((NPAD, NUM_CLASSES), f32))(
            part2, part2, b2r)

    return out[:N_NODES]


# speedup vs baseline: 1.0728x; 1.0728x over previous
import functools

import jax
import jax.numpy as jnp
from jax import lax
from jax.experimental import pallas as pl
from jax.experimental.pallas import tpu as pltpu
from jax.experimental.pallas import tpu_sc as plsc

N_NODES = 10000
N_EDGES = 320000
NUM_FEATURES = 128
NUM_CLASSES = 10
HID = 8
IN_HEAD = 8

NPAD = 10240
BN = 256
NBLK = NPAD // BN
DUMMY = 10016

E_TOT = N_EDGES + N_NODES
NW = 32
UNROLL = 4
RPT = NPAD // 16
KSUB1, SG1 = 64, 1
KSUB2, SG2 = 96, 2


def _chunks_per_worker(k):
    return UNROLL * (-(-E_TOT // (k * NW * UNROLL)))

W1CAT = 80
W2CAT = 16


def _tc_proj1_body(x_ref, w_ref, as_ref, ad_ref, o_ref, o2_ref):
    h = jnp.dot(x_ref[...], w_ref[...], preferred_element_type=jnp.float32)
    a_src = jnp.dot(h, as_ref[...], preferred_element_type=jnp.float32)
    a_dst = jnp.dot(h, ad_ref[...], preferred_element_type=jnp.float32)
    z8 = jnp.zeros((BN, 8), jnp.float32)
    o_ref[...] = jnp.concatenate([h, a_src, z8], axis=1)
    o2_ref[...] = jnp.concatenate([a_dst, z8], axis=1)


def _tc_mid_body(p0_ref, p1_ref, e8_ref, b1_ref, w2e_ref, o_ref):
    acc = p0_ref[0] + p1_ref[0]
    num = acc[:, :64]
    den = jnp.dot(acc[:, 64:72], e8_ref[...],
                  preferred_element_type=jnp.float32)
    h1 = num / (den + 1e-16) + b1_ref[...]
    h1 = jnp.where(h1 > 0, h1, jnp.exp(jnp.minimum(h1, 0.0)) - 1.0)
    o_ref[...] = jnp.dot(h1, w2e_ref[...], preferred_element_type=jnp.float32)


def _tc_out_body(p0_ref, p1_ref, b2_ref, o_ref):
    acc = p0_ref[0] + p1_ref[0]
    z = acc[:, :10] / (acc[:, 10:11] + 1e-16) + b2_ref[...]
    m = jnp.max(z, axis=1, keepdims=True)
    zs = z - m
    o_ref[...] = zs - jnp.log(jnp.sum(jnp.exp(zs), axis=1, keepdims=True))


def _sc_edge_pass(ec_hbm, srows_hbm, atab_hbm, zeros_hbm, out_hbm,
                  accum, ebuf, atab, rows, orows, isem, tsem, grsem, ssem,
                  *, width, n_feat, n_head, asrc_col, bf16_tab, ksub, sg,
                  ch_per_w):
    cid = lax.axis_index("c")
    sid = lax.axis_index("s")
    wid = cid * 16 + sid
    row0 = sid * RPT
    ch = n_feat // n_head
    k = ksub * sg

    tcopy = pltpu.async_copy(atab_hbm, atab, tsem)
    pltpu.sync_copy(zeros_hbm, accum.at[pl.ds(row0, RPT)])
    tcopy.wait()
    plsc.subcore_barrier()

    zv = jnp.zeros((16,), jnp.float32)
    for b in range(2):
        for g in range(k // 16):
            eis = lax.iota(jnp.int32, 16) + g * 16
            for col in range(n_feat + n_head, width):
                plsc.store_scatter(
                    orows.at[b], [eis, jnp.full((16,), col, jnp.int32)], zv)

    def fetch(j, q):
        pltpu.async_copy(ec_hbm.at[wid * ch_per_w + j], ebuf.at[q],
                         isem.at[q])

    def wait_fetch(q):
        pltpu.make_async_copy(ec_hbm.at[0], ebuf.at[q], isem.at[q]).wait()

    def gather(b, q):
        for s in range(sg):
            pltpu.async_copy(srows_hbm.at[ebuf.at[q, 0, s]],
                             rows.at[b, pl.ds(s * ksub, ksub)], grsem.at[b])

    def wait_gather(b):
        for s in range(sg):
            pltpu.make_async_copy(srows_hbm.at[ebuf.at[0, 0, s]],
                                  rows.at[b, pl.ds(s * ksub, ksub)],
                                  grsem.at[b]).wait()

    def scatter(b, q):
        for s in range(sg):
            pltpu.async_copy(orows.at[b, pl.ds(s * ksub, ksub)],
                             accum.at[ebuf.at[q, 1, s]], ssem.at[b],
                             add=True)

    def wait_scatter(b):
        for s in range(sg):
            pltpu.make_async_copy(orows.at[b, pl.ds(s * ksub, ksub)],
                                  accum.at[ebuf.at[0, 1, s]],
                                  ssem.at[b]).wait()

    def compute(b, q):
        rb = rows.at[b]
        ob = orows.at[b]
        for s_ in range(sg):

            def group_body(g, inner, s_=s_):
                ei = lax.iota(jnp.int32, 16) + (s_ * ksub + g * 16)
                dstv = ebuf[q, 1, s_, pl.ds(g * 16, 16)]
                if bf16_tab:
                    dvs = []
                    dstv4 = dstv * (n_head // 2)
                    for pc in range(n_head // 2):
                        pv = plsc.load_gather(atab, [dstv4 + pc])
                        d0, d1 = plsc.unpack(
                            plsc.bitcast(pv, jnp.bfloat16),
                            format=plsc.PackFormat.INTERLEAVED,
                            preferred_element_type=jnp.float32)
                        dvs += [d0, d1]
                else:
                    dvs = [plsc.load_gather(atab, [dstv])]
                ws = []
                for hd in range(n_head):
                    av = plsc.load_gather(
                        rb, [ei, jnp.full((16,), asrc_col + hd, jnp.int32)])
                    s = av + dvs[hd]
                    w = jnp.exp(jnp.maximum(s, s * 0.2))
                    plsc.store_scatter(
                        ob, [ei, jnp.full((16,), n_feat + hd, jnp.int32)], w)
                    ws.append(w)
                for col in range(n_feat):
                    hv = plsc.load_gather(
                        rb, [ei, jnp.full((16,), col, jnp.int32)])
                    plsc.store_scatter(
                        ob, [ei, jnp.full((16,), col, jnp.int32)],
                        hv * ws[col // ch])
                return inner

            lax.fori_loop(0, ksub // 16, group_body, 0)

    def step(j, jq, with_scatter_wait):
        b = jq % 2
        q = jq % 4
        wait_gather(b)
        if with_scatter_wait:
            wait_scatter(b)
        fetch(j + 2, (jq + 2) % 4)
        wait_fetch((jq + 1) % 4)
        gather((b + 1) % 2, (jq + 1) % 4)
        compute(b, q)
        scatter(b, q)

    fetch(0, 0)
    fetch(1, 1)
    wait_fetch(0)
    gather(0, 0)
    for j in range(2):
        step(j, j, with_scatter_wait=False)
    for j in range(2, UNROLL):
        step(j, j, with_scatter_wait=True)

    def loop_body(m, carry):
        j0 = m * UNROLL
        for r in range(UNROLL):
            step(j0 + r, r, with_scatter_wait=True)
        return carry

    lax.fori_loop(1, ch_per_w // UNROLL, loop_body, 0)

    wait_scatter(0)
    wait_scatter(1)
    wait_gather(ch_per_w % 2)
    wait_fetch((ch_per_w + 1) % 4)
    plsc.subcore_barrier()
    pltpu.sync_copy(accum.at[pl.ds(row0, RPT)],
                    out_hbm.at[cid, pl.ds(row0, RPT)])


@functools.cache
def _make_sc_pass(width, n_feat, n_head, asrc_col, bf16_tab, ksub, sg):
    mesh = plsc.VectorSubcoreMesh(core_axis_name="c", subcore_axis_name="s")
    ch_per_w = _chunks_per_worker(ksub * sg)
    body = functools.partial(_sc_edge_pass, width=width, n_feat=n_feat,
                             n_head=n_head, asrc_col=asrc_col,
                             bf16_tab=bf16_tab, ksub=ksub, sg=sg,
                             ch_per_w=ch_per_w)
    k = ksub * sg
    tab_t = (pltpu.VMEM((NPAD * n_head // 2,), jnp.int32) if bf16_tab
             else pltpu.VMEM((NPAD,), jnp.float32))
    return pl.kernel(
        body,
        out_type=jax.ShapeDtypeStruct((2, NPAD, width), jnp.float32),
        mesh=mesh,
        compiler_params=pltpu.CompilerParams(
            needs_layout_passes=False, use_tc_tiling_on_sc=False),
        scratch_types=[
            pltpu.VMEM_SHARED((NPAD, width), jnp.float32),
            pltpu.VMEM((4, 2, sg, ksub), jnp.int32),
            tab_t,
            pltpu.VMEM((2, k, width), jnp.float32),
            pltpu.VMEM((2, k, width), jnp.float32),
            pltpu.SemaphoreType.DMA((4,)),
            pltpu.SemaphoreType.DMA,
            pltpu.SemaphoreType.DMA((2,)),
            pltpu.SemaphoreType.DMA((2,)),
        ],
    )


def _tc_call(body, grid, in_specs, out_specs, out_shape):
    return pl.pallas_call(body, grid=grid, in_specs=in_specs,
                          out_specs=out_specs, out_shape=out_shape)


def kernel(x, edge_index, W1, att_src1, att_dst1, b1, W2, att_src2,
           att_dst2, b2):
    f32 = jnp.float32

    loop = jnp.arange(N_NODES, dtype=jnp.int32)

    def make_ec(ksub, sg):
        k = ksub * sg
        nch_alloc = _chunks_per_worker(k) * NW + 2
        padi = jnp.full((nch_alloc * k - E_TOT,), DUMMY, jnp.int32)
        s = jnp.concatenate([edge_index[0].astype(jnp.int32), loop, padi])
        d = jnp.concatenate([edge_index[1].astype(jnp.int32), loop, padi])
        return jnp.stack([s.reshape(nch_alloc, sg, ksub),
                          d.reshape(nch_alloc, sg, ksub)], axis=1)

    ec1 = make_ec(KSUB1, SG1)
    ec2 = make_ec(KSUB2, SG2)

    ar = jnp.arange(IN_HEAD * HID)
    As = jnp.zeros((IN_HEAD * HID, IN_HEAD), f32).at[
        ar, ar // HID].set(att_src1.reshape(-1))
    Ad = jnp.zeros((IN_HEAD * HID, IN_HEAD), f32).at[
        ar, ar // HID].set(att_dst1.reshape(-1))
    E8 = jnp.zeros((IN_HEAD, IN_HEAD * HID), f32).at[
        ar // HID, ar].set(1.0)
    W2e = jnp.concatenate([
        W2, W2 @ att_src2.reshape(NUM_CLASSES, 1),
        W2 @ att_dst2.reshape(NUM_CLASSES, 1),
        jnp.zeros((IN_HEAD * HID, 4), f32)], axis=1)

    xp = jnp.zeros((NPAD, NUM_FEATURES), f32).at[:N_NODES].set(x)

    full = lambda shp: pl.BlockSpec(shp, lambda i: (0, 0))
    blk = lambda w: pl.BlockSpec((BN, w), lambda i: (i, 0))
    srows1, drows1 = _tc_call(
        _tc_proj1_body, (NBLK,),
        [blk(NUM_FEATURES), full((NUM_FEATURES, 64)), full((64, 8)),
         full((64, 8))],
        [blk(W1CAT), blk(16)],
        [jax.ShapeDtypeStruct((NPAD, W1CAT), f32),
         jax.ShapeDtypeStruct((NPAD, 16), f32)])(xp, W1, As, Ad)

    adst_bf = drows1[:, :IN_HEAD].astype(jnp.bfloat16)
    adst_pairs = jax.lax.bitcast_convert_type(
        adst_bf.reshape(NPAD, IN_HEAD // 2, 2), jnp.int32).reshape(-1)
    zeros1 = jnp.zeros((RPT, W1CAT), f32)
    part1 = _make_sc_pass(W1CAT, 64, IN_HEAD, 64, True, KSUB1, SG1)(
        ec1, srows1, adst_pairs, zeros1)

    b1r = b1.reshape(1, 64)
    rows2 = _tc_call(
        _tc_mid_body, (NBLK,),
        [pl.BlockSpec((1, BN, W1CAT), lambda i: (0, i, 0)),
         pl.BlockSpec((1, BN, W1CAT), lambda i: (1, i, 0)),
         full((IN_HEAD, 64)), full((1, 64)), full((64, W2CAT))],
        blk(W2CAT),
        jax.ShapeDtypeStruct((NPAD, W2CAT), f32))(
            part1, part1, E8, b1r, W2e)

    adst2 = rows2[:, 11]
    zeros2 = jnp.zeros((RPT, W2CAT), f32)
    part2 = _make_sc_pass(W2CAT, 10, 1, 10, False, KSUB2, SG2)(
        ec2, rows2, adst2, zeros2)

    b2r = b2.reshape(1, NUM_CLASSES)
    out = _tc_call(
        _tc_out_body, (NBLK,),
        [pl.BlockSpec((1, BN, W2CAT), lambda i: (0, i, 0)),
         pl.BlockSpec((1, BN, W2CAT), lambda i: (1, i, 0)),
         full((1, NUM_CLASSES))],
        blk(NUM_CLASSES),
        jax.ShapeDtypeStruct((NPAD, NUM_CLASSES), f32))(
            part2, part2, b2r)

    return out[:N_NODES]


# speedup vs baseline: 1.1284x; 1.0519x over previous
import functools

import jax
import jax.numpy as jnp
from jax import lax
from jax.experimental import pallas as pl
from jax.experimental.pallas import tpu as pltpu
from jax.experimental.pallas import tpu_sc as plsc

N_NODES = 10000
N_EDGES = 320000
NUM_FEATURES = 128
NUM_CLASSES = 10
HID = 8
IN_HEAD = 8

NPAD = 10240
BN = 256
NBLK = NPAD // BN
DUMMY = 10016

E_TOT = N_EDGES + N_NODES
NW = 32
UNROLL = 4
RPT = NPAD // 16
KSUB1, SG1 = 96, 1
KSUB2, SG2 = 96, 2


def _chunks_per_worker(k):
    return UNROLL * (-(-E_TOT // (k * NW * UNROLL)))

W1CAT = 80
W2CAT = 16


def _tc_proj1_body(x_ref, w_ref, as_ref, ad_ref, o_ref, o2_ref):
    h = jnp.dot(x_ref[...], w_ref[...], preferred_element_type=jnp.float32)
    a_src = jnp.dot(h, as_ref[...], preferred_element_type=jnp.float32)
    a_dst = jnp.dot(h, ad_ref[...], preferred_element_type=jnp.float32)
    z8 = jnp.zeros((BN, 8), jnp.float32)
    o_ref[...] = jnp.concatenate([h, a_src, z8], axis=1)
    o2_ref[...] = jnp.concatenate([a_dst, z8], axis=1)


def _tc_mid_body(p0_ref, p1_ref, e8_ref, b1_ref, w2e_ref, o_ref):
    acc = p0_ref[0] + p1_ref[0]
    num = acc[:, :64]
    den = jnp.dot(acc[:, 64:72], e8_ref[...],
                  preferred_element_type=jnp.float32)
    h1 = num / (den + 1e-16) + b1_ref[...]
    h1 = jnp.where(h1 > 0, h1, jnp.exp(jnp.minimum(h1, 0.0)) - 1.0)
    o_ref[...] = jnp.dot(h1, w2e_ref[...], preferred_element_type=jnp.float32)


def _tc_out_body(p0_ref, p1_ref, b2_ref, o_ref):
    acc = p0_ref[0] + p1_ref[0]
    z = acc[:, :10] / (acc[:, 10:11] + 1e-16) + b2_ref[...]
    m = jnp.max(z, axis=1, keepdims=True)
    zs = z - m
    o_ref[...] = zs - jnp.log(jnp.sum(jnp.exp(zs), axis=1, keepdims=True))


def _sc_edge_pass(ec_hbm, srows_hbm, atab_hbm, zeros_hbm, out_hbm,
                  accum, ebuf, atab, rows, orows, isem, tsem, grsem, ssem,
                  *, width, n_feat, n_head, asrc_col, bf16_tab, ksub, sg,
                  ch_per_w):
    cid = lax.axis_index("c")
    sid = lax.axis_index("s")
    wid = cid * 16 + sid
    row0 = sid * RPT
    ch = n_feat // n_head
    k = ksub * sg

    tcopy = pltpu.async_copy(atab_hbm, atab, tsem)
    pltpu.sync_copy(zeros_hbm, accum.at[pl.ds(row0, RPT)])
    tcopy.wait()
    plsc.subcore_barrier()

    zv = jnp.zeros((16,), jnp.float32)
    for b in range(2):
        for g in range(k // 16):
            eis = lax.iota(jnp.int32, 16) + g * 16
            for col in range(n_feat + n_head, width):
                plsc.store_scatter(
                    orows.at[b], [eis, jnp.full((16,), col, jnp.int32)], zv)

    def fetch(j, q):
        pltpu.async_copy(ec_hbm.at[wid * ch_per_w + j], ebuf.at[q],
                         isem.at[q])

    def wait_fetch(q):
        pltpu.make_async_copy(ec_hbm.at[0], ebuf.at[q], isem.at[q]).wait()

    def gather(b, q):
        for s in range(sg):
            pltpu.async_copy(srows_hbm.at[ebuf.at[q, 0, s]],
                             rows.at[b, pl.ds(s * ksub, ksub)], grsem.at[b])

    def wait_gather(b):
        for s in range(sg):
            pltpu.make_async_copy(srows_hbm.at[ebuf.at[0, 0, s]],
                                  rows.at[b, pl.ds(s * ksub, ksub)],
                                  grsem.at[b]).wait()

    def scatter(b, q):
        for s in range(sg):
            pltpu.async_copy(orows.at[b, pl.ds(s * ksub, ksub)],
                             accum.at[ebuf.at[q, 1, s]], ssem.at[b],
                             add=True)

    def wait_scatter(b):
        for s in range(sg):
            pltpu.make_async_copy(orows.at[b, pl.ds(s * ksub, ksub)],
                                  accum.at[ebuf.at[0, 1, s]],
                                  ssem.at[b]).wait()

    def compute(b, q):
        rb = rows.at[b]
        ob = orows.at[b]
        for s_ in range(sg):

            def group_body(g, inner, s_=s_):
                ei = lax.iota(jnp.int32, 16) + (s_ * ksub + g * 16)
                dstv = ebuf[q, 1, s_, pl.ds(g * 16, 16)]
                if bf16_tab:
                    dvs = []
                    dstv4 = dstv * (n_head // 2)
                    for pc in range(n_head // 2):
                        pv = plsc.load_gather(atab, [dstv4 + pc])
                        d0, d1 = plsc.unpack(
                            plsc.bitcast(pv, jnp.bfloat16),
                            format=plsc.PackFormat.INTERLEAVED,
                            preferred_element_type=jnp.float32)
                        dvs += [d0, d1]
                else:
                    dvs = [plsc.load_gather(atab, [dstv])]
                ws = []
                for hd in range(n_head):
                    av = plsc.load_gather(
                        rb, [ei, jnp.full((16,), asrc_col + hd, jnp.int32)])
                    s = av + dvs[hd]
                    w = jnp.exp(jnp.maximum(s, s * 0.2))
                    plsc.store_scatter(
                        ob, [ei, jnp.full((16,), n_feat + hd, jnp.int32)], w)
                    ws.append(w)
                for col in range(n_feat):
                    hv = plsc.load_gather(
                        rb, [ei, jnp.full((16,), col, jnp.int32)])
                    plsc.store_scatter(
                        ob, [ei, jnp.full((16,), col, jnp.int32)],
                        hv * ws[col // ch])
                return inner

            lax.fori_loop(0, ksub // 16, group_body, 0)

    def step(j, jq, with_scatter_wait):
        b = jq % 2
        q = jq % 4
        wait_gather(b)
        if with_scatter_wait:
            wait_scatter(b)
        fetch(j + 2, (jq + 2) % 4)
        wait_fetch((jq + 1) % 4)
        gather((b + 1) % 2, (jq + 1) % 4)
        compute(b, q)
        scatter(b, q)

    fetch(0, 0)
    fetch(1, 1)
    wait_fetch(0)
    gather(0, 0)
    for j in range(2):
        step(j, j, with_scatter_wait=False)
    for j in range(2, UNROLL):
        step(j, j, with_scatter_wait=True)

    def loop_body(m, carry):
        j0 = m * UNROLL
        for r in range(UNROLL):
            step(j0 + r, r, with_scatter_wait=True)
        return carry

    lax.fori_loop(1, ch_per_w // UNROLL, loop_body, 0)

    wait_scatter(0)
    wait_scatter(1)
    wait_gather(ch_per_w % 2)
    wait_fetch((ch_per_w + 1) % 4)
    plsc.subcore_barrier()
    pltpu.sync_copy(accum.at[pl.ds(row0, RPT)],
                    out_hbm.at[cid, pl.ds(row0, RPT)])


@functools.cache
def _make_sc_pass(width, n_feat, n_head, asrc_col, bf16_tab, ksub, sg):
    mesh = plsc.VectorSubcoreMesh(core_axis_name="c", subcore_axis_name="s")
    ch_per_w = _chunks_per_worker(ksub * sg)
    body = functools.partial(_sc_edge_pass, width=width, n_feat=n_feat,
                             n_head=n_head, asrc_col=asrc_col,
                             bf16_tab=bf16_tab, ksub=ksub, sg=sg,
                             ch_per_w=ch_per_w)
    k = ksub * sg
    tab_t = (pltpu.VMEM((NPAD * n_head // 2,), jnp.int32) if bf16_tab
             else pltpu.VMEM((NPAD,), jnp.float32))
    return pl.kernel(
        body,
        out_type=jax.ShapeDtypeStruct((2, NPAD, width), jnp.float32),
        mesh=mesh,
        compiler_params=pltpu.CompilerParams(
            needs_layout_passes=False, use_tc_tiling_on_sc=False),
        scratch_types=[
            pltpu.VMEM_SHARED((NPAD, width), jnp.float32),
            pltpu.VMEM((4, 2, sg, ksub), jnp.int32),
            tab_t,
            pltpu.VMEM((2, k, width), jnp.float32),
            pltpu.VMEM((2, k, width), jnp.float32),
            pltpu.SemaphoreType.DMA((4,)),
            pltpu.SemaphoreType.DMA,
            pltpu.SemaphoreType.DMA((2,)),
            pltpu.SemaphoreType.DMA((2,)),
        ],
    )


def _tc_call(body, grid, in_specs, out_specs, out_shape):
    return pl.pallas_call(body, grid=grid, in_specs=in_specs,
                          out_specs=out_specs, out_shape=out_shape)


def kernel(x, edge_index, W1, att_src1, att_dst1, b1, W2, att_src2,
           att_dst2, b2):
    f32 = jnp.float32

    loop = jnp.arange(N_NODES, dtype=jnp.int32)

    def make_ec(ksub, sg):
        k = ksub * sg
        nch_alloc = _chunks_per_worker(k) * NW + 2
        padi = jnp.full((nch_alloc * k - E_TOT,), DUMMY, jnp.int32)
        s = jnp.concatenate([edge_index[0].astype(jnp.int32), loop, padi])
        d = jnp.concatenate([edge_index[1].astype(jnp.int32), loop, padi])
        return jnp.stack([s.reshape(nch_alloc, sg, ksub),
                          d.reshape(nch_alloc, sg, ksub)], axis=1)

    ec1 = make_ec(KSUB1, SG1)
    ec2 = make_ec(KSUB2, SG2)

    ar = jnp.arange(IN_HEAD * HID)
    As = jnp.zeros((IN_HEAD * HID, IN_HEAD), f32).at[
        ar, ar // HID].set(att_src1.reshape(-1))
    Ad = jnp.zeros((IN_HEAD * HID, IN_HEAD), f32).at[
        ar, ar // HID].set(att_dst1.reshape(-1))
    E8 = jnp.zeros((IN_HEAD, IN_HEAD * HID), f32).at[
        ar // HID, ar].set(1.0)
    W2e = jnp.concatenate([
        W2, W2 @ att_src2.reshape(NUM_CLASSES, 1),
        W2 @ att_dst2.reshape(NUM_CLASSES, 1),
        jnp.zeros((IN_HEAD * HID, 4), f32)], axis=1)

    xp = jnp.zeros((NPAD, NUM_FEATURES), f32).at[:N_NODES].set(x)

    full = lambda shp: pl.BlockSpec(shp, lambda i: (0, 0))
    blk = lambda w: pl.BlockSpec((BN, w), lambda i: (i, 0))
    srows1, drows1 = _tc_call(
        _tc_proj1_body, (NBLK,),
        [blk(NUM_FEATURES), full((NUM_FEATURES, 64)), full((64, 8)),
         full((64, 8))],
        [blk(W1CAT), blk(16)],
        [jax.ShapeDtypeStruct((NPAD, W1CAT), f32),
         jax.ShapeDtypeStruct((NPAD, 16), f32)])(xp, W1, As, Ad)

    adst_bf = drows1[:, :IN_HEAD].astype(jnp.bfloat16)
    adst_pairs = jax.lax.bitcast_convert_type(
        adst_bf.reshape(NPAD, IN_HEAD // 2, 2), jnp.int32).reshape(-1)
    zeros1 = jnp.zeros((RPT, W1CAT), f32)
    part1 = _make_sc_pass(W1CAT, 64, IN_HEAD, 64, True, KSUB1, SG1)(
        ec1, srows1, adst_pairs, zeros1)

    b1r = b1.reshape(1, 64)
    rows2 = _tc_call(
        _tc_mid_body, (NBLK,),
        [pl.BlockSpec((1, BN, W1CAT), lambda i: (0, i, 0)),
         pl.BlockSpec((1, BN, W1CAT), lambda i: (1, i, 0)),
         full((IN_HEAD, 64)), full((1, 64)), full((64, W2CAT))],
        blk(W2CAT),
        jax.ShapeDtypeStruct((NPAD, W2CAT), f32))(
            part1, part1, E8, b1r, W2e)

    adst2 = rows2[:, 11]
    zeros2 = jnp.zeros((RPT, W2CAT), f32)
    part2 = _make_sc_pass(W2CAT, 10, 1, 10, False, KSUB2, SG2)(
        ec2, rows2, adst2, zeros2)

    b2r = b2.reshape(1, NUM_CLASSES)
    out = _tc_call(
        _tc_out_body, (NBLK,),
        [pl.BlockSpec((1, BN, W2CAT), lambda i: (0, i, 0)),
         pl.BlockSpec((1, BN, W2CAT), lambda i: (1, i, 0)),
         full((1, NUM_CLASSES))],
        blk(NUM_CLASSES),
        jax.ShapeDtypeStruct((NPAD, NUM_CLASSES), f32))(
            part2, part2, b2r)

    return out[:N_NODES]
